# baseline (speedup 1.0000x reference)
import functools

import jax
import jax.numpy as jnp
from jax.experimental import pallas as pl

N_E = 2048
E_DIM = 32
P = 8
BETA = 0.25


def _vq_block_kernel(z_ref, e_ref, zq_ref, loss_ref):
    hw = z_ref.shape[2] * z_ref.shape[3]
    zb = z_ref[0].reshape(E_DIM, hw)
    zf = zb.T
    e = e_ref[...]

    z_sq = jnp.sum(zf * zf, axis=1, keepdims=True)
    e_sq = jnp.sum(e * e, axis=1)
    mm = jax.lax.dot_general(zf, e, (((1,), (1,)), ((), ())))
    d = (z_sq + e_sq[None, :]) - 2.0 * mm

    dmin = jnp.min(d, axis=1, keepdims=True)
    iota = jax.lax.broadcasted_iota(jnp.int32, d.shape, 1)
    cand = jnp.where(d == dmin, iota, N_E)
    idx = jnp.min(cand, axis=1, keepdims=True)

    onehot = (iota == idx).astype(jnp.float32)
    zq = jax.lax.dot_general(
        onehot, e, (((1,), (0,)), ((), ())),
        precision=jax.lax.Precision.HIGHEST)

    part = jnp.sum((zq - zf) ** 2).reshape(1, 1)
    step = pl.program_id(0)

    @pl.when(step == 0)
    def _init():
        loss_ref[...] = jnp.zeros((1, 1), jnp.float32)

    loss_ref[...] += part

    zq_t = zq.T.reshape(z_ref.shape)
    zin = z_ref[...]
    zq_ref[...] = zin + (zq_t - zin)


@functools.partial(jax.jit, static_argnames=())
def kernel(z, embedding_weight):
    b, c, h, w = z.shape
    new_c = c // P
    grid = b * P

    zq, loss_sum = pl.pallas_call(
        _vq_block_kernel,
        grid=(grid,),
        in_specs=[
            pl.BlockSpec((1, new_c, h, w), lambda i: (i // P, i % P, 0, 0)),
            pl.BlockSpec((N_E, E_DIM), lambda i: (0, 0)),
        ],
        out_specs=[
            pl.BlockSpec((1, new_c, h, w), lambda i: (i // P, i % P, 0, 0)),
            pl.BlockSpec((1, 1), lambda i: (0, 0)),
        ],
        out_shape=[
            jax.ShapeDtypeStruct(z.shape, jnp.float32),
            jax.ShapeDtypeStruct((1, 1), jnp.float32),
        ],
    )(z, embedding_weight)

    n_el = b * c * h * w
    m = loss_sum[0, 0] / n_el
    loss = m + BETA * m
    return (zq, loss)


# speedup vs baseline: 2.8828x; 2.8828x over previous
import functools

import jax
import jax.numpy as jnp
from jax.experimental import pallas as pl
from jax.experimental.pallas import tpu as pltpu

N_E = 2048
E_DIM = 32
P = 8
BETA = 0.25
HI = 16
LO = 128


def _vq_block_kernel(z_ref, e_ref, e2t_ref, zq_ref, loss_ref, esq_ref):
    hw = z_ref.shape[2] * z_ref.shape[3]
    zb = z_ref[0].reshape(E_DIM, hw)
    e = e_ref[...]
    step = pl.program_id(0)

    @pl.when(step == 0)
    def _init():
        esq_ref[...] = jnp.sum(e * e, axis=1, keepdims=True)
        loss_ref[...] = jnp.zeros((1, 1), jnp.float32)

    e_sq = esq_ref[...]
    z_sq = jnp.sum(zb * zb, axis=0, keepdims=True)
    zb2 = zb + zb
    mm2 = jax.lax.dot_general(
        e, zb2, (((1,), (0,)), ((), ())))
    d = (z_sq + e_sq) - mm2

    dmin = jnp.min(d, axis=0, keepdims=True)
    riota = jax.lax.broadcasted_iota(jnp.int32, (N_E, hw), 0)
    cand = jnp.where(d == dmin, riota, N_E)
    idx = jnp.min(cand, axis=0, keepdims=True)

    lo = idx & (LO - 1)
    hi = idx >> 7
    liota = jax.lax.broadcasted_iota(jnp.int32, (LO, hw), 0)
    ohlo = jnp.where(liota == lo, 1.0, 0.0).astype(jnp.bfloat16)
    t = jax.lax.dot_general(
        e2t_ref[...], ohlo, (((1,), (0,)), ((), ())),
        preferred_element_type=jnp.float32)
    siota = jax.lax.broadcasted_iota(jnp.int32, (HI * E_DIM, hw), 0) >> 5
    pm = jnp.where(siota == hi, t, 0.0)
    acc = pm[0:E_DIM, :]
    for h in range(1, HI):
        acc = acc + pm[h * E_DIM:(h + 1) * E_DIM, :]

    diff = acc - zb
    loss_ref[...] += jnp.sum(diff * diff).reshape(1, 1)
    zq_ref[...] = (zb + diff).reshape(z_ref.shape)


@functools.partial(jax.jit, static_argnames=())
def kernel(z, embedding_weight):
    b, c, h, w = z.shape
    new_c = c // P
    grid = b * P

    e2t = (embedding_weight.reshape(HI, LO, E_DIM)
           .transpose(0, 2, 1)
           .reshape(HI * E_DIM, LO)
           .astype(jnp.bfloat16))

    zq, loss_sum = pl.pallas_call(
        _vq_block_kernel,
        grid=(grid,),
        in_specs=[
            pl.BlockSpec((1, new_c, h, w), lambda i: (i // P, i % P, 0, 0)),
            pl.BlockSpec((N_E, E_DIM), lambda i: (0, 0)),
            pl.BlockSpec((HI * E_DIM, LO), lambda i: (0, 0)),
        ],
        out_specs=[
            pl.BlockSpec((1, new_c, h, w), lambda i: (i // P, i % P, 0, 0)),
            pl.BlockSpec((1, 1), lambda i: (0, 0)),
        ],
        out_shape=[
            jax.ShapeDtypeStruct(z.shape, jnp.float32),
            jax.ShapeDtypeStruct((1, 1), jnp.float32),
        ],
        scratch_shapes=[pltpu.VMEM((N_E, 1), jnp.float32)],
    )(z, embedding_weight, e2t)

    n_el = b * c * h * w
    m = loss_sum[0, 0] / n_el
    loss = m + BETA * m
    return (zq, loss)


# speedup vs baseline: 3.4264x; 1.1886x over previous
import functools

import jax
import jax.numpy as jnp
from jax.experimental import pallas as pl
from jax.experimental.pallas import tpu as pltpu

N_E = 2048
E_DIM = 32
P = 8
BETA = 0.25
HI = 16
LO = 128


def _vq_block_kernel(z_ref, e_ref, e2t_ref, zq_ref, loss_ref, e2_ref, esq_ref):
    hw = z_ref.shape[2]
    zb = z_ref[0]
    step = pl.program_id(0)

    @pl.when(step == 0)
    def _init():
        e = e_ref[...]
        e2_ref[...] = e + e
        esq_ref[...] = jnp.sum(e * e, axis=1, keepdims=True)
        loss_ref[...] = jnp.zeros((1, 1), jnp.float32)

    e_sq = esq_ref[...]
    z_sq = jnp.sum(zb * zb, axis=0, keepdims=True)
    mm2 = jax.lax.dot_general(
        e2_ref[...], zb, (((1,), (0,)), ((), ())))
    d = (z_sq + e_sq) - mm2

    dmin = jnp.min(d, axis=0, keepdims=True)
    riota = jax.lax.broadcasted_iota(jnp.int32, (N_E, hw), 0)
    cand = jnp.where(d == dmin, riota, N_E)
    idx = jnp.min(cand, axis=0, keepdims=True)

    lo = idx & (LO - 1)
    hi = idx >> 7
    liota = jax.lax.broadcasted_iota(jnp.int32, (LO, hw), 0)
    ohlo = jnp.where(liota == lo, 1.0, 0.0).astype(jnp.bfloat16)
    t = jax.lax.dot_general(
        e2t_ref[...], ohlo, (((1,), (0,)), ((), ())),
        preferred_element_type=jnp.float32)
    siota = jax.lax.broadcasted_iota(jnp.int32, (HI * E_DIM, hw), 0) >> 5
    pm = jnp.where(siota == hi, t, 0.0)
    acc = pm[0:E_DIM, :]
    for h in range(1, HI):
        acc = acc + pm[h * E_DIM:(h + 1) * E_DIM, :]

    diff = acc - zb
    loss_ref[...] += jnp.sum(diff * diff).reshape(1, 1)
    zq_ref[...] = (zb + diff).reshape(z_ref.shape)


@functools.partial(jax.jit, static_argnames=())
def kernel(z, embedding_weight):
    b, c, h, w = z.shape
    new_c = c // P
    hw = h * w
    grid = b * P

    e2t = (embedding_weight.reshape(HI, LO, E_DIM)
           .transpose(0, 2, 1)
           .reshape(HI * E_DIM, LO)
           .astype(jnp.bfloat16))
    z3 = z.reshape(b, c, hw)

    zq, loss_sum = pl.pallas_call(
        _vq_block_kernel,
        grid=(grid,),
        in_specs=[
            pl.BlockSpec((1, new_c, hw), lambda i: (i // P, i % P, 0)),
            pl.BlockSpec((N_E, E_DIM), lambda i: (0, 0)),
            pl.BlockSpec((HI * E_DIM, LO), lambda i: (0, 0)),
        ],
        out_specs=[
            pl.BlockSpec((1, new_c, hw), lambda i: (i // P, i % P, 0)),
            pl.BlockSpec((1, 1), lambda i: (0, 0)),
        ],
        out_shape=[
            jax.ShapeDtypeStruct((b, c, hw), jnp.float32),
            jax.ShapeDtypeStruct((1, 1), jnp.float32),
        ],
        scratch_shapes=[
            pltpu.VMEM((N_E, E_DIM), jnp.float32),
            pltpu.VMEM((N_E, 1), jnp.float32),
        ],
    )(z3, embedding_weight, e2t)

    n_el = b * c * h * w
    m = loss_sum[0, 0] / n_el
    loss = m + BETA * m
    return (zq.reshape(b, c, h, w), loss)


# speedup vs baseline: 3.8234x; 1.1159x over previous
import functools

import jax
import jax.numpy as jnp
from jax.experimental import pallas as pl
from jax.experimental.pallas import tpu as pltpu

N_E = 2048
E_DIM = 32
P = 8
BETA = 0.25
HI = 16
LO = 128
GPB = 2


def _vq_block_kernel(z_ref, e_ref, e2t_ref, zq_ref, loss_ref, e2_ref, esq_ref):
    hw = z_ref.shape[2]
    cols = GPB * hw
    zbf = z_ref[0]
    zb = jnp.concatenate(
        [zbf[g * E_DIM:(g + 1) * E_DIM, :] for g in range(GPB)], axis=1)
    step = pl.program_id(0)

    @pl.when(step == 0)
    def _init():
        e = e_ref[...]
        e2_ref[...] = e + e
        esq_ref[...] = jnp.sum(e * e, axis=1, keepdims=True)
        loss_ref[...] = jnp.zeros((1, 1), jnp.float32)

    e_sq = esq_ref[...]
    z_sq = jnp.sum(zb * zb, axis=0, keepdims=True)
    mm2 = jax.lax.dot_general(
        e2_ref[...], zb, (((1,), (0,)), ((), ())))
    d = (z_sq + e_sq) - mm2

    dmin = jnp.min(d, axis=0, keepdims=True)
    riota = jax.lax.broadcasted_iota(jnp.int32, (N_E, cols), 0)
    cand = jnp.where(d == dmin, riota, N_E)
    idx = jnp.min(cand, axis=0, keepdims=True)

    lo = idx & (LO - 1)
    hi = idx >> 7
    liota = jax.lax.broadcasted_iota(jnp.int32, (LO, cols), 0)
    ohlo = jnp.where(liota == lo, 1.0, 0.0).astype(jnp.bfloat16)
    t = jax.lax.dot_general(
        e2t_ref[...], ohlo, (((1,), (0,)), ((), ())),
        preferred_element_type=jnp.float32)
    siota = jax.lax.broadcasted_iota(jnp.int32, (HI * E_DIM, cols), 0) >> 5
    pm = jnp.where(siota == hi, t, 0.0)
    acc = pm[0:E_DIM, :]
    for h in range(1, HI):
        acc = acc + pm[h * E_DIM:(h + 1) * E_DIM, :]

    diff = acc - zb
    loss_ref[...] += jnp.sum(diff * diff).reshape(1, 1)
    out = zb + diff
    zq_ref[...] = jnp.concatenate(
        [out[:, g * hw:(g + 1) * hw] for g in range(GPB)],
        axis=0).reshape(z_ref.shape)


@functools.partial(jax.jit, static_argnames=())
def kernel(z, embedding_weight):
    b, c, h, w = z.shape
    new_c = c // P
    hw = h * w
    grid = (b * P) // GPB

    e2t = (embedding_weight.reshape(HI, LO, E_DIM)
           .transpose(0, 2, 1)
           .reshape(HI * E_DIM, LO)
           .astype(jnp.bfloat16))
    z3 = z.reshape(b, c, hw)
    blocks_per_b = P // GPB

    zq, loss_sum = pl.pallas_call(
        _vq_block_kernel,
        grid=(grid,),
        in_specs=[
            pl.BlockSpec((1, GPB * new_c, hw),
                         lambda i: (i // blocks_per_b, i % blocks_per_b, 0)),
            pl.BlockSpec((N_E, E_DIM), lambda i: (0, 0)),
            pl.BlockSpec((HI * E_DIM, LO), lambda i: (0, 0)),
        ],
        out_specs=[
            pl.BlockSpec((1, GPB * new_c, hw),
                         lambda i: (i // blocks_per_b, i % blocks_per_b, 0)),
            pl.BlockSpec((1, 1), lambda i: (0, 0)),
        ],
        out_shape=[
            jax.ShapeDtypeStruct((b, c, hw), jnp.float32),
            jax.ShapeDtypeStruct((1, 1), jnp.float32),
        ],
        scratch_shapes=[
            pltpu.VMEM((N_E, E_DIM), jnp.float32),
            pltpu.VMEM((N_E, 1), jnp.float32),
        ],
    )(z3, embedding_weight, e2t)

    n_el = b * c * h * w
    m = loss_sum[0, 0] / n_el
    loss = m + BETA * m
    return (zq.reshape(b, c, h, w), loss)
